# Initial kernel scaffold; baseline (speedup 1.0000x reference)
import jax
import jax.numpy as jnp
from jax.experimental import pallas as pl


def kernel(x, pos, batch, params):
    raise NotImplementedError("write your pallas kernel here")


# speedup vs baseline: 1.2935x; 1.2935x over previous
import functools

import jax
import jax.numpy as jnp
from jax import lax
from jax.experimental import pallas as pl

_B = 8
_N = 2048
_F = 9
_MAXN = 64


def _fps_body(posT_ref, qT_ref, *, n, m):
    iota = lax.broadcasted_iota(jnp.int32, (_B, n), 1)
    px = posT_ref[0]
    py = posT_ref[1]
    pz = posT_ref[2]
    cur0 = posT_ref[:, :, 0]
    qT_ref[0] = cur0
    mind0 = jnp.full((_B, n), jnp.inf, jnp.float32)

    def body(i, carry):
        mind, cur = carry
        dx = px - cur[0][:, None]
        dy = py - cur[1][:, None]
        dz = pz - cur[2][:, None]
        d = dx * dx + dy * dy + dz * dz
        mind = jnp.minimum(mind, d)
        mx = jnp.max(mind, axis=1, keepdims=True)
        nxt = jnp.min(jnp.where(mind >= mx, iota, n), axis=1)
        oh = (iota == nxt[:, None]).astype(jnp.float32)
        cur = jnp.stack(
            [
                jnp.sum(px * oh, axis=1),
                jnp.sum(py * oh, axis=1),
                jnp.sum(pz * oh, axis=1),
            ]
        )
        qT_ref[pl.ds(i, 1)] = cur[None]
        return mind, cur

    lax.fori_loop(1, m, body, (mind0, cur0))


def _fps_centroids(pos_b, m):
    n = pos_b.shape[1]
    posT = jnp.transpose(pos_b, (2, 0, 1))
    qT = pl.pallas_call(
        functools.partial(_fps_body, n=n, m=m),
        out_shape=jax.ShapeDtypeStruct((m, 3, _B), jnp.float32),
    )(posT)
    return jnp.transpose(qT, (2, 0, 1))


def _mlp(params, h):
    for W, b in params:
        h = jax.nn.relu(h @ W + b)
    return h


def _gather_nbrs(arr, idx):
    return jax.vmap(lambda a, i: a[i])(arr, idx)


def _sa_stage(nn_params, x, pos, q, r):
    d2 = jnp.sum((q[:, :, None, :] - pos[:, None, :, :]) ** 2, axis=-1)
    neg = jnp.where(d2 <= r * r, -d2, -jnp.inf)
    vals, nbr = jax.lax.top_k(neg, _MAXN)
    valid = vals > -jnp.inf
    xn = _gather_nbrs(x, nbr)
    pn = _gather_nbrs(pos, nbr)
    rel = pn - q[:, :, None, :]
    feat = jnp.concatenate([xn, rel], axis=-1)
    h = _mlp(nn_params, feat)
    h = jnp.where(valid[..., None], h, -jnp.inf)
    return jnp.max(h, axis=2)


def _knn_interp(xs, ps, pt, k):
    d2 = jnp.sum((pt[:, :, None, :] - ps[:, None, :, :]) ** 2, axis=-1)
    kk = min(k, ps.shape[1])
    neg, idx = jax.lax.top_k(-d2, kk)
    w = 1.0 / (-neg + 1e-8)
    w = w / jnp.sum(w, axis=-1, keepdims=True)
    xg = _gather_nbrs(xs, idx)
    return jnp.sum(w[..., None] * xg, axis=2)


def kernel(x, pos, batch, params):
    xb = x.reshape(_B, _N, _F)
    pb = pos.reshape(_B, _N, 3)

    q1 = _fps_centroids(pb, _N // 4)
    x1 = _sa_stage(params["sa1"], xb, pb, q1, 0.2)
    q2 = _fps_centroids(q1, _N // 16)
    x2 = _sa_stage(params["sa2"], x1, q1, q2, 0.4)

    h3 = _mlp(params["sa3"], jnp.concatenate([x2, q2], axis=-1))
    x3 = jnp.max(h3, axis=1, keepdims=True)
    p3 = jnp.zeros((_B, 1, 3), jnp.float32)

    xi = _knn_interp(x3, p3, q2, 1)
    h = _mlp(params["fp3"], jnp.concatenate([xi, x2], axis=-1))
    xi = _knn_interp(h, q2, q1, 3)
    h = _mlp(params["fp2"], jnp.concatenate([xi, x1], axis=-1))
    xi = _knn_interp(h, q1, pb, 3)
    h = _mlp(params["fp1"], jnp.concatenate([xi, xb], axis=-1))
    h = h.reshape(_B * _N, 128)
    W1, b1 = params["lin1"]
    W2, b2 = params["lin2"]
    h = jax.nn.relu(h @ W1 + b1)
    return h @ W2 + b2


# speedup vs baseline: 2.6256x; 2.0299x over previous
import functools

import jax
import jax.numpy as jnp
import numpy as np
from jax import lax
from jax.experimental import pallas as pl

_B = 8
_N = 2048
_F = 9
_K = 64

_dot = functools.partial(jnp.dot, preferred_element_type=jnp.float32)


def _relu(v):
    return jnp.maximum(v, 0.0)


def _fps_body(posT_ref, qT_ref, *, n, m):
    iota = lax.broadcasted_iota(jnp.int32, (_B, n), 1)
    px = posT_ref[0]
    py = posT_ref[1]
    pz = posT_ref[2]
    cur0 = posT_ref[:, :, 0]
    qT_ref[0] = cur0
    mind0 = jnp.full((_B, n), jnp.inf, jnp.float32)

    def body(i, carry):
        mind, cur = carry
        dx = px - cur[0][:, None]
        dy = py - cur[1][:, None]
        dz = pz - cur[2][:, None]
        d = dx * dx + dy * dy + dz * dz
        mind = jnp.minimum(mind, d)
        mx = jnp.max(mind, axis=1, keepdims=True)
        nxt = jnp.min(jnp.where(mind >= mx, iota, n), axis=1)
        oh = (iota == nxt[:, None]).astype(jnp.float32)
        cur = jnp.stack(
            [
                jnp.sum(px * oh, axis=1),
                jnp.sum(py * oh, axis=1),
                jnp.sum(pz * oh, axis=1),
            ]
        )
        qT_ref[pl.ds(i, 1)] = cur[None]
        return mind, cur

    lax.fori_loop(1, m, body, (mind0, cur0))


def _fps_centroids(pos_b, m):
    n = pos_b.shape[1]
    posT = jnp.transpose(pos_b, (2, 0, 1))
    qT = pl.pallas_call(
        functools.partial(_fps_body, n=n, m=m),
        out_shape=jax.ShapeDtypeStruct((m, 3, _B), jnp.float32),
    )(posT)
    return jnp.transpose(qT, (2, 0, 1))


def _sa_mlp_body(feat_ref, valid_ref, q_ref, w1_ref, b1_ref, w1p_ref,
                 w2_ref, b2_ref, w3_ref, b3_ref, out_ref, *, rb):
    f = feat_ref[...]
    c1 = w1_ref.shape[1]
    h1 = _dot(f, w1_ref[...]) + b1_ref[...]
    qw = _dot(q_ref[...], w1p_ref[...])
    h1 = h1.reshape(rb, _K, c1) - qw[:, None, :]
    h1 = _relu(h1).reshape(rb * _K, c1)
    h2 = _relu(_dot(h1, w2_ref[...]) + b2_ref[...])
    h3 = _relu(_dot(h2, w3_ref[...]) + b3_ref[...])
    c3 = w3_ref.shape[1]
    h3 = h3.reshape(rb, _K, c3)
    v = valid_ref[...]
    h3 = jnp.where(v[:, :, None] > 0.0, h3, -jnp.inf)
    out_ref[...] = jnp.max(h3, axis=1)


def _sa_mlp(feat, valid, q, params, rb):
    (w1, b1), (w2, b2), (w3, b3) = params
    r = valid.shape[0]
    d = feat.shape[1]
    c1, c2, c3 = w1.shape[1], w2.shape[1], w3.shape[1]
    w1pad = jnp.zeros((d, c1), jnp.float32).at[: w1.shape[0]].set(w1)
    w1p = w1[-3:]
    grid = (r // rb,)
    return pl.pallas_call(
        functools.partial(_sa_mlp_body, rb=rb),
        grid=grid,
        in_specs=[
            pl.BlockSpec((rb * _K, d), lambda i: (i, 0)),
            pl.BlockSpec((rb, _K), lambda i: (i, 0)),
            pl.BlockSpec((rb, 3), lambda i: (i, 0)),
            pl.BlockSpec((d, c1), lambda i: (0, 0)),
            pl.BlockSpec((1, c1), lambda i: (0, 0)),
            pl.BlockSpec((3, c1), lambda i: (0, 0)),
            pl.BlockSpec((c1, c2), lambda i: (0, 0)),
            pl.BlockSpec((1, c2), lambda i: (0, 0)),
            pl.BlockSpec((c2, c3), lambda i: (0, 0)),
            pl.BlockSpec((1, c3), lambda i: (0, 0)),
        ],
        out_specs=pl.BlockSpec((rb, c3), lambda i: (i, 0)),
        out_shape=jax.ShapeDtypeStruct((r, c3), jnp.float32),
    )(feat, valid, q, w1pad, b1[None], w1p, w2, b2[None], w3, b3[None])


def _sa3fp3_body(g_ref, a1_ref, a1b_ref, a2_ref, a2b_ref, a3_ref, a3b_ref,
                 f1t_ref, f1b_ref, f1bias_ref, f2_ref, f2bias_ref, out_ref):
    g = g_ref[0]
    h = _relu(_dot(g, a1_ref[...]) + a1b_ref[...])
    h = _relu(_dot(h, a2_ref[...]) + a2b_ref[...])
    h = _relu(_dot(h, a3_ref[...]) + a3b_ref[...])
    x3 = jnp.max(h, axis=0, keepdims=True)
    t = _dot(x3, f1t_ref[...])
    x2 = g[:, :256]
    h = _relu(_dot(x2, f1b_ref[...]) + t + f1bias_ref[...])
    h = _relu(_dot(h, f2_ref[...]) + f2bias_ref[...])
    out_ref[0] = h


def _sa3fp3(x2q2, p_sa3, p_fp3):
    (a1, a1b), (a2, a2b), (a3, a3b) = p_sa3
    (f1, f1bias), (f2, f2bias) = p_fp3
    f1t, f1b = f1[:1024], f1[1024:]
    spec_w = lambda w: pl.BlockSpec(w.shape, lambda b: (0,) * w.ndim)
    args = (a1, a1b[None], a2, a2b[None], a3, a3b[None],
            f1t, f1b, f1bias[None], f2, f2bias[None])
    return pl.pallas_call(
        _sa3fp3_body,
        grid=(_B,),
        in_specs=[pl.BlockSpec((1, 128, 259), lambda b: (b, 0, 0))]
        + [spec_w(w) for w in args],
        out_specs=pl.BlockSpec((1, 128, 256), lambda b: (b, 0, 0)),
        out_shape=jax.ShapeDtypeStruct((_B, 128, 256), jnp.float32),
    )(x2q2, *args)


def _knn3_weights(pt, psT, ns):
    nt = pt.shape[0]
    d2 = jnp.zeros((nt, ns), jnp.float32)
    for c in range(3):
        dc = pt[:, c : c + 1] - psT[c : c + 1, :]
        d2 = d2 + dc * dc
    neg = -d2
    iota = lax.broadcasted_iota(jnp.int32, (nt, ns), 1)
    ws, idxs = [], []
    for _ in range(3):
        mx = jnp.max(neg, axis=1, keepdims=True)
        ix = jnp.min(jnp.where(neg >= mx, iota, ns), axis=1, keepdims=True)
        ws.append(1.0 / (-mx + 1e-8))
        idxs.append(ix)
        neg = jnp.where(iota == ix, -jnp.inf, neg)
    wsum = ws[0] + ws[1] + ws[2]
    mat = jnp.zeros((nt, ns), jnp.float32)
    for k in range(3):
        mat = mat + (ws[k] / wsum) * (iota == idxs[k]).astype(jnp.float32)
    return mat


def _fp2_body(q1_ref, q2T_ref, hs_ref, x1_ref, v1t_ref, v1b_ref, v1bias_ref,
              v2_ref, v2bias_ref, out_ref):
    mat = _knn3_weights(q1_ref[0], q2T_ref[0], 128)
    xi = _dot(mat, hs_ref[0])
    h = _relu(_dot(xi, v1t_ref[...]) + _dot(x1_ref[0], v1b_ref[...])
              + v1bias_ref[...])
    h = _relu(_dot(h, v2_ref[...]) + v2bias_ref[...])
    out_ref[0] = h


def _fp2(q1, q2T, hfp3, x1, p_fp2):
    (v1, v1bias), (v2, v2bias) = p_fp2
    v1t, v1b = v1[:256], v1[256:]
    spec_w = lambda w: pl.BlockSpec(w.shape, lambda b: (0,) * w.ndim)
    args = (v1t, v1b, v1bias[None], v2, v2bias[None])
    return pl.pallas_call(
        _fp2_body,
        grid=(_B,),
        in_specs=[
            pl.BlockSpec((1, 512, 3), lambda b: (b, 0, 0)),
            pl.BlockSpec((1, 3, 128), lambda b: (b, 0, 0)),
            pl.BlockSpec((1, 128, 256), lambda b: (b, 0, 0)),
            pl.BlockSpec((1, 512, 128), lambda b: (b, 0, 0)),
        ]
        + [spec_w(w) for w in args],
        out_specs=pl.BlockSpec((1, 512, 128), lambda b: (b, 0, 0)),
        out_shape=jax.ShapeDtypeStruct((_B, 512, 128), jnp.float32),
    )(q1, q2T, hfp3, x1, *args)


def _fp1_body(pb_ref, q1T_ref, hs_ref, xb_ref, u1x_ref, u1p_ref, u1b_ref,
              u2_ref, u2b_ref, u3_ref, u3b_ref, l1_ref, l1b_ref, l2_ref,
              l2b_ref, out_ref):
    mat = _knn3_weights(pb_ref[0], q1T_ref[0], 512)
    xi = _dot(mat, hs_ref[0])
    h = _relu(_dot(xi, u1x_ref[...]) + _dot(xb_ref[0], u1p_ref[...])
              + u1b_ref[...])
    h = _relu(_dot(h, u2_ref[...]) + u2b_ref[...])
    h = _relu(_dot(h, u3_ref[...]) + u3b_ref[...])
    h = _relu(_dot(h, l1_ref[...]) + l1b_ref[...])
    out_ref[0] = _dot(h, l2_ref[...]) + l2b_ref[...]


def _fp1(pb, q1T, hfp2, xb, p_fp1, lin1, lin2):
    (u1, u1bias), (u2, u2b), (u3, u3b) = p_fp1
    w1, b1 = lin1
    w2, b2 = lin2
    u1x, u1p = u1[:128], u1[128:]
    spec_w = lambda w: pl.BlockSpec(w.shape, lambda b: (0,) * w.ndim)
    args = (u1x, u1p, u1bias[None], u2, u2b[None], u3, u3b[None],
            w1, b1[None], w2, b2[None])
    return pl.pallas_call(
        _fp1_body,
        grid=(_B,),
        in_specs=[
            pl.BlockSpec((1, 2048, 3), lambda b: (b, 0, 0)),
            pl.BlockSpec((1, 3, 512), lambda b: (b, 0, 0)),
            pl.BlockSpec((1, 512, 128), lambda b: (b, 0, 0)),
            pl.BlockSpec((1, 2048, _F), lambda b: (b, 0, 0)),
        ]
        + [spec_w(w) for w in args],
        out_specs=pl.BlockSpec((1, 2048, 3), lambda b: (b, 0, 0)),
        out_shape=jax.ShapeDtypeStruct((_B, 2048, 3), jnp.float32),
    )(pb, q1T, hfp2, xb, *args)


def _group_interim(xf, posf, q, r, n, m, dpad):
    d2 = jnp.sum((q[:, :, None, :] - posf[:, None, :, :]) ** 2, axis=-1)
    neg = jnp.where(d2 <= r * r, -d2, -jnp.inf)
    vals, nbr = jax.lax.top_k(neg, _K)
    valid = (vals > -jnp.inf).astype(jnp.float32)
    tab = jnp.concatenate(
        [xf, posf, jnp.zeros(posf.shape[:2] + (dpad - xf.shape[-1] - 3,),
                             jnp.float32)], axis=-1)
    feat = jax.vmap(lambda a, i: a[i])(tab, nbr)
    return feat.reshape(-1, dpad), valid.reshape(-1, _K)


def kernel(x, pos, batch, params):
    xb = x.reshape(_B, _N, _F)
    pb = pos.reshape(_B, _N, 3)

    q1 = _fps_centroids(pb, _N // 4)
    feat1, valid1 = _group_interim(xb, pb, q1, 0.2, _N, _N // 4, 16)
    x1 = _sa_mlp(feat1, valid1, q1.reshape(-1, 3), params["sa1"], 256)
    x1b = x1.reshape(_B, _N // 4, 128)

    q2 = _fps_centroids(q1, _N // 16)
    feat2, valid2 = _group_interim(x1b, q1, q2, 0.4, _N // 4, _N // 16, 144)
    x2 = _sa_mlp(feat2, valid2, q2.reshape(-1, 3), params["sa2"], 128)
    x2b = x2.reshape(_B, _N // 16, 256)

    x2q2 = jnp.concatenate([x2b, q2], axis=-1)
    hfp3 = _sa3fp3(x2q2, params["sa3"], params["fp3"])

    q2T = jnp.transpose(q2, (0, 2, 1))
    hfp2 = _fp2(q1, q2T, hfp3, x1b, params["fp2"])

    q1T = jnp.transpose(q1, (0, 2, 1))
    out = _fp1(pb, q1T, hfp2, xb, params["fp1"], params["lin1"],
               params["lin2"])
    return out.reshape(_B * _N, 3)
